# Initial kernel scaffold; baseline (speedup 1.0000x reference)
import jax
import jax.numpy as jnp
from jax.experimental import pallas as pl


def kernel(source, weight):
    raise NotImplementedError("write your pallas kernel here")


# speedup vs baseline: 6.3333x; 6.3333x over previous
import functools

import jax
import jax.numpy as jnp
from jax import lax
from jax.experimental import pallas as pl
from jax.experimental.pallas import tpu as pltpu
from jax.experimental.pallas import tpu_sc as plsc

_NUM_CORES = 2
_NUM_SUBCORES = 16
_NW = _NUM_CORES * _NUM_SUBCORES
_CHUNK = 128


def _emb_call(total, V, D, chunks):
    mesh = plsc.VectorSubcoreMesh(core_axis_name="c", subcore_axis_name="s")
    per_w = chunks * _CHUNK

    @functools.partial(
        pl.kernel,
        mesh=mesh,
        out_type=jax.ShapeDtypeStruct((total, D), jnp.float32),
        scratch_types=[
            pltpu.VMEM((chunks, _CHUNK), jnp.int32),
            pltpu.VMEM((_CHUNK, D), jnp.float32),
            pltpu.SemaphoreType.DMA,
        ],
    )
    def emb(idx_hbm, table_hbm, out_hbm, idx_v, rows_v, sem):
        wid = lax.axis_index("s") * _NUM_CORES + lax.axis_index("c")
        base = wid * per_w
        pltpu.sync_copy(idx_hbm.at[wid], idx_v)

        def body(j, carry):
            pltpu.async_copy(table_hbm.at[idx_v.at[j]], rows_v, sem).wait()
            pltpu.sync_copy(rows_v, out_hbm.at[pl.ds(base + j * _CHUNK, _CHUNK)])
            return carry

        lax.fori_loop(0, chunks, body, 0)

    return emb


def kernel(source, weight):
    B, H = source.shape
    V, D = weight.shape
    total = B * H
    assert total % (_NW * _CHUNK) == 0
    per_w = total // _NW
    chunks = per_w // _CHUNK
    idx3 = source.reshape(_NW, chunks, _CHUNK).astype(jnp.int32)
    out = _emb_call(total, V, D, chunks)(idx3, weight)
    return out.reshape(B, H, D)


# speedup vs baseline: 9.1936x; 1.4516x over previous
import functools

import jax
import jax.numpy as jnp
from jax import lax
from jax.experimental import pallas as pl
from jax.experimental.pallas import tpu as pltpu
from jax.experimental.pallas import tpu_sc as plsc

_NUM_CORES = 2
_NUM_SUBCORES = 16
_NW = _NUM_CORES * _NUM_SUBCORES
_CHUNK = 128
_NBUF = 4


def _emb_call(total, V, D, chunks):
    mesh = plsc.VectorSubcoreMesh(core_axis_name="c", subcore_axis_name="s")
    per_w = chunks * _CHUNK
    groups = chunks // _NBUF

    @functools.partial(
        pl.kernel,
        mesh=mesh,
        out_type=jax.ShapeDtypeStruct((total, D), jnp.float32),
        scratch_types=[pltpu.VMEM((chunks, _CHUNK), jnp.int32)]
        + [pltpu.VMEM((_CHUNK, D), jnp.float32) for _ in range(_NBUF)]
        + [pltpu.SemaphoreType.DMA for _ in range(2 * _NBUF)],
    )
    def emb(idx_hbm, table_hbm, out_hbm, idx_v, *bufs):
        rows = bufs[:_NBUF]
        gsem = bufs[_NBUF : 2 * _NBUF]
        ssem = bufs[2 * _NBUF :]
        wid = lax.axis_index("s") * _NUM_CORES + lax.axis_index("c")
        base = wid * per_w
        pltpu.sync_copy(idx_hbm.at[wid], idx_v)

        def gather_d(j, b):
            return pltpu.make_async_copy(table_hbm.at[idx_v.at[j]], rows[b], gsem[b])

        def store_d(j, b):
            return pltpu.make_async_copy(
                rows[b], out_hbm.at[pl.ds(base + j * _CHUNK, _CHUNK)], ssem[b]
            )

        for b in range(_NBUF):
            gather_d(b, b).start()

        def group(g, carry):
            for b in range(_NBUF):
                j = g * _NBUF + b
                gather_d(j, b).wait()
                store_d(j, b).start()
            for b in range(_NBUF):
                j = g * _NBUF + b
                store_d(j, b).wait()
                gather_d(j + _NBUF, b).start()
            return carry

        lax.fori_loop(0, groups - 1, group, 0)

        g_last = groups - 1
        for b in range(_NBUF):
            j = g_last * _NBUF + b
            gather_d(j, b).wait()
            store_d(j, b).start()
        for b in range(_NBUF):
            j = g_last * _NBUF + b
            store_d(j, b).wait()

    return emb


def kernel(source, weight):
    B, H = source.shape
    V, D = weight.shape
    total = B * H
    assert total % (_NW * _CHUNK * _NBUF) == 0
    per_w = total // _NW
    chunks = per_w // _CHUNK
    idx3 = source.reshape(_NW, chunks, _CHUNK).astype(jnp.int32)
    out = _emb_call(total, V, D, chunks)(idx3, weight)
    return out.reshape(B, H, D)


# speedup vs baseline: 9.2020x; 1.0009x over previous
import functools

import jax
import jax.numpy as jnp
from jax import lax
from jax.experimental import pallas as pl
from jax.experimental.pallas import tpu as pltpu
from jax.experimental.pallas import tpu_sc as plsc

_NUM_CORES = 2
_NUM_SUBCORES = 16
_NW = _NUM_CORES * _NUM_SUBCORES
_CHUNK = 128
_NBUF = 5


def _emb_call(total, V, D, chunks):
    mesh = plsc.VectorSubcoreMesh(core_axis_name="c", subcore_axis_name="s")
    per_w = chunks * _CHUNK
    groups = chunks // _NBUF

    @functools.partial(
        pl.kernel,
        mesh=mesh,
        out_type=jax.ShapeDtypeStruct((total, D), jnp.float32),
        scratch_types=[pltpu.VMEM((chunks, _CHUNK), jnp.int32)]
        + [pltpu.VMEM((_CHUNK, D), jnp.float32) for _ in range(_NBUF)]
        + [pltpu.SemaphoreType.DMA for _ in range(2 * _NBUF)],
    )
    def emb(idx_hbm, table_hbm, out_hbm, idx_v, *bufs):
        rows = bufs[:_NBUF]
        gsem = bufs[_NBUF : 2 * _NBUF]
        ssem = bufs[2 * _NBUF :]
        wid = lax.axis_index("s") * _NUM_CORES + lax.axis_index("c")
        base = wid * per_w
        pltpu.sync_copy(idx_hbm.at[wid], idx_v)

        def gather_d(j, b):
            return pltpu.make_async_copy(table_hbm.at[idx_v.at[j]], rows[b], gsem[b])

        def store_d(j, b):
            return pltpu.make_async_copy(
                rows[b], out_hbm.at[pl.ds(base + j * _CHUNK, _CHUNK)], ssem[b]
            )

        for b in range(_NBUF):
            gather_d(b, b).start()

        def group(g, carry):
            for b in range(_NBUF):
                j = g * _NBUF + b
                gather_d(j, b).wait()
                store_d(j, b).start()
            for b in range(_NBUF):
                j = g * _NBUF + b
                store_d(j, b).wait()
                gather_d(j + _NBUF, b).start()
            return carry

        lax.fori_loop(0, groups - 1, group, 0)

        g_last = groups - 1
        for b in range(_NBUF):
            j = g_last * _NBUF + b
            gather_d(j, b).wait()
            store_d(j, b).start()
        for b in range(_NBUF):
            j = g_last * _NBUF + b
            store_d(j, b).wait()

    return emb


def kernel(source, weight):
    B, H = source.shape
    V, D = weight.shape
    total = B * H
    assert total % (_NW * _CHUNK * _NBUF) == 0
    per_w = total // _NW
    chunks = per_w // _CHUNK
    idx3 = source.reshape(_NW, chunks, _CHUNK).astype(jnp.int32)
    out = _emb_call(total, V, D, chunks)(idx3, weight)
    return out.reshape(B, H, D)
